# baseline (speedup 1.0000x reference)
import jax
import jax.numpy as jnp
from jax import lax
from jax.experimental import pallas as pl
from jax.experimental.pallas import tpu as pltpu
from jax.experimental.pallas import tpu_sc as plsc

N_USERS = 1000000
N_ITEMS = 1000000
D = 64
B = 16384

NC = 2
NS = 16
NW = NC * NS
BPW = B // NW
CHUNK = 128
NCHUNK = BPW // CHUNK
R = 7
NSUP = BPW // R
REM = BPW - NSUP * R
SLAB = 128


def _sc_body(u3_hbm, i3_hbm, Ut_hbm, Vt_hbm, ub_hbm, vb_hbm,
             gb_hbm, out_hbm,
             uflat_v, iflat_v, uslab_v, vslab_v,
             ubr_v, vbr_v, out_v, gb_v, sem, s0, s1, s2, s3, s4, s5, s6):
    wid = lax.axis_index("s") * NC + lax.axis_index("c")
    base = wid * BPW

    pltpu.sync_copy(u3_hbm.at[wid], uflat_v)
    pltpu.sync_copy(i3_hbm.at[wid], iflat_v)
    pltpu.sync_copy(gb_hbm, gb_v)
    gb = gb_v[...]
    lanes0 = jnp.arange(16, dtype=jnp.int32)
    mask128 = ~jnp.int32(127)
    sems = [s0, s1, s2, s3, s4, s5, s6]

    def fire(l, j):
        uvec = uflat_v[pl.ds((l // 16) * 16, 16)]
        ivec = iflat_v[pl.ds((l // 16) * 16, 16)]
        onel = jnp.where(lanes0 == l % 16, jnp.int32(1), jnp.int32(0))
        su = pl.multiple_of(jnp.sum(uvec * onel) & mask128, 128)
        si = pl.multiple_of(jnp.sum(ivec * onel) & mask128, 128)
        pltpu.async_copy(Ut_hbm.at[:, pl.ds(su, SLAB)], uslab_v.at[j], sems[j])
        pltpu.async_copy(Vt_hbm.at[:, pl.ds(si, SLAB)], vslab_v.at[j], sems[j])

    for j in range(R):
        fire(jnp.int32(j), j)

    bcopies = []
    for c in range(NCHUNK):
        rows = pl.ds(c * CHUNK, CHUNK)
        bcopies.append(pltpu.async_copy(ub_hbm.at[uflat_v.at[rows]], ubr_v.at[rows], sem))
        bcopies.append(pltpu.async_copy(vb_hbm.at[iflat_v.at[rows]], vbr_v.at[rows], sem))
    for cp in bcopies:
        cp.wait()

    def consume(l, j, acc, may_fire):
        pltpu.make_async_copy(
            Ut_hbm.at[:, pl.ds(0, SLAB)], uslab_v.at[j], sems[j]).wait()
        pltpu.make_async_copy(
            Vt_hbm.at[:, pl.ds(0, SLAB)], vslab_v.at[j], sems[j]).wait()

        uvec = uflat_v[pl.ds((l // 16) * 16, 16)]
        ivec = iflat_v[pl.ds((l // 16) * 16, 16)]
        lane = l % 16
        onel = jnp.where(lanes0 == lane, jnp.int32(1), jnp.int32(0))
        cu = jnp.full((16,), jnp.sum((uvec & 127) * onel), jnp.int32)
        cv = jnp.full((16,), jnp.sum((ivec & 127) * onel), jnp.int32)
        jv = jnp.full((16,), j, jnp.int32)
        psum = jnp.zeros((16,), jnp.float32)
        for c in range(D // 16):
            dvec = lanes0 + c * 16
            psum = psum + (plsc.load_gather(uslab_v, [jv, dvec, cu])
                           * plsc.load_gather(vslab_v, [jv, dvec, cv]))
        acc = jnp.where(lanes0 == lane, jnp.sum(psum), acc)

        if may_fire:
            @pl.when(l + R < BPW)
            def _():
                fire(l + R, j)

        is15 = lane == 15

        @pl.when(is15)
        def _():
            jj = l // 16
            out_v[pl.ds(jj * 16, 16)] = (acc + ubr_v[pl.ds(jj * 16, 16)]
                                         + vbr_v[pl.ds(jj * 16, 16)] + gb)
        return jnp.where(jnp.full((16,), is15), jnp.zeros((16,), jnp.float32), acc)

    def super_iter(k, acc):
        for j in range(R):
            acc = consume(k * R + j, j, acc, True)
        return acc

    acc = lax.fori_loop(0, NSUP, super_iter, jnp.zeros((16,), jnp.float32))
    for j in range(REM):
        acc = consume(jnp.int32(NSUP * R + j), j, acc, False)

    pltpu.sync_copy(out_v, out_hbm.at[pl.ds(base, BPW)])


@jax.jit
def _mf_scores(u3, i3, Ut, Vt, ubf, vbf, gb1):
    mesh = plsc.VectorSubcoreMesh(core_axis_name="c", subcore_axis_name="s")
    kern = pl.kernel(
        _sc_body,
        out_type=jax.ShapeDtypeStruct((B,), jnp.float32),
        mesh=mesh,
        compiler_params=pltpu.CompilerParams(
            needs_layout_passes=False, use_tc_tiling_on_sc=True),
        scratch_types=[
            pltpu.VMEM((BPW,), jnp.int32),
            pltpu.VMEM((BPW,), jnp.int32),
            pltpu.VMEM((R, D, SLAB), jnp.float32),
            pltpu.VMEM((R, D, SLAB), jnp.float32),
            pltpu.VMEM((BPW,), jnp.float32),
            pltpu.VMEM((BPW,), jnp.float32),
            pltpu.VMEM((BPW,), jnp.float32),
            pltpu.VMEM((16,), jnp.float32),
        ] + [pltpu.SemaphoreType.DMA] * (1 + R),
    )
    return kern(u3, i3, Ut, Vt, ubf, vbf, gb1)


def kernel(u, i, U, V, ub, vb, gb):
    perm = jnp.argsort(u)
    u = jnp.zeros_like(u).at[perm].set(u[perm])
    u3 = u.reshape(NW, BPW)
    i3 = i.reshape(NW, BPW)
    gb1 = jnp.broadcast_to(jnp.asarray(gb, jnp.float32), (16,))
    return _mf_scores(u3, i3, U.T, V.T,
                      ub.reshape(N_USERS), vb.reshape(N_ITEMS), gb1)


# speedup vs baseline: 1.1475x; 1.1475x over previous
import jax
import jax.numpy as jnp
from jax import lax
from jax.experimental import pallas as pl
from jax.experimental.pallas import tpu as pltpu
from jax.experimental.pallas import tpu_sc as plsc

N_USERS = 1000000
N_ITEMS = 1000000
D = 64
B = 16384

NC = 2
NS = 16
NW = NC * NS
BPW = B // NW
CHUNK = 128
NCHUNK = BPW // CHUNK
R = 7
NSUP = BPW // R
REM = BPW - NSUP * R
SLAB = 128


def _sc_body(u3_hbm, i3_hbm, Ut_hbm, Vt_hbm, ub_hbm, vb_hbm,
             gb_hbm, out_hbm,
             uflat_v, iflat_v, uslab_v, vslab_v,
             ubr_v, vbr_v, out_v, gb_v, sem, s0, s1, s2, s3, s4, s5, s6):
    wid = lax.axis_index("s") * NC + lax.axis_index("c")
    base = wid * BPW

    pltpu.sync_copy(u3_hbm.at[wid], uflat_v)
    pltpu.sync_copy(i3_hbm.at[wid], iflat_v)
    pltpu.sync_copy(gb_hbm, gb_v)
    gb = gb_v[...]
    lanes0 = jnp.arange(16, dtype=jnp.int32)
    mask128 = ~jnp.int32(127)
    sems = [s0, s1, s2, s3, s4, s5, s6]

    def fire(l, j):
        uvec = uflat_v[pl.ds((l // 16) * 16, 16)]
        ivec = iflat_v[pl.ds((l // 16) * 16, 16)]
        onel = jnp.where(lanes0 == l % 16, jnp.int32(1), jnp.int32(0))
        su = pl.multiple_of(jnp.sum(uvec * onel) & mask128, 128)
        si = pl.multiple_of(jnp.sum(ivec * onel) & mask128, 128)
        pltpu.async_copy(Ut_hbm.at[:, pl.ds(su, SLAB)], uslab_v.at[j], sems[j])
        pltpu.async_copy(Vt_hbm.at[:, pl.ds(si, SLAB)], vslab_v.at[j], sems[j])

    for j in range(R):
        fire(jnp.int32(j), j)

    bcopies = []
    for c in range(NCHUNK):
        rows = pl.ds(c * CHUNK, CHUNK)
        bcopies.append(pltpu.async_copy(ub_hbm.at[uflat_v.at[rows]], ubr_v.at[rows], sem))
        bcopies.append(pltpu.async_copy(vb_hbm.at[iflat_v.at[rows]], vbr_v.at[rows], sem))
    for cp in bcopies:
        cp.wait()

    def consume(l, j, acc, may_fire):
        pltpu.make_async_copy(
            Ut_hbm.at[:, pl.ds(0, SLAB)], uslab_v.at[j], sems[j]).wait()
        pltpu.make_async_copy(
            Vt_hbm.at[:, pl.ds(0, SLAB)], vslab_v.at[j], sems[j]).wait()

        uvec = uflat_v[pl.ds((l // 16) * 16, 16)]
        ivec = iflat_v[pl.ds((l // 16) * 16, 16)]
        lane = l % 16
        onel = jnp.where(lanes0 == lane, jnp.int32(1), jnp.int32(0))
        cu = jnp.full((16,), jnp.sum((uvec & 127) * onel), jnp.int32)
        cv = jnp.full((16,), jnp.sum((ivec & 127) * onel), jnp.int32)
        jv = jnp.full((16,), j, jnp.int32)
        psum = jnp.zeros((16,), jnp.float32)
        for c in range(D // 16):
            dvec = lanes0 + c * 16
            psum = psum + (plsc.load_gather(uslab_v, [jv, dvec, cu])
                           * plsc.load_gather(vslab_v, [jv, dvec, cv]))
        acc = jnp.where(lanes0 == lane, jnp.sum(psum), acc)

        if may_fire:
            @pl.when(l + R < BPW)
            def _():
                fire(l + R, j)

        is15 = lane == 15

        @pl.when(is15)
        def _():
            jj = l // 16
            out_v[pl.ds(jj * 16, 16)] = (acc + ubr_v[pl.ds(jj * 16, 16)]
                                         + vbr_v[pl.ds(jj * 16, 16)] + gb)
        return jnp.where(jnp.full((16,), is15), jnp.zeros((16,), jnp.float32), acc)

    def super_iter(k, acc):
        for j in range(R):
            acc = consume(k * R + j, j, acc, True)
        return acc

    acc = lax.fori_loop(0, NSUP, super_iter, jnp.zeros((16,), jnp.float32))
    for j in range(REM):
        acc = consume(jnp.int32(NSUP * R + j), j, acc, False)

    pltpu.sync_copy(out_v, out_hbm.at[pl.ds(base, BPW)])


@jax.jit
def _mf_scores(u3, i3, Ut, Vt, ubf, vbf, gb1):
    mesh = plsc.VectorSubcoreMesh(core_axis_name="c", subcore_axis_name="s")
    kern = pl.kernel(
        _sc_body,
        out_type=jax.ShapeDtypeStruct((B,), jnp.float32),
        mesh=mesh,
        compiler_params=pltpu.CompilerParams(
            needs_layout_passes=False, use_tc_tiling_on_sc=True),
        scratch_types=[
            pltpu.VMEM((BPW,), jnp.int32),
            pltpu.VMEM((BPW,), jnp.int32),
            pltpu.VMEM((R, D, SLAB), jnp.float32),
            pltpu.VMEM((R, D, SLAB), jnp.float32),
            pltpu.VMEM((BPW,), jnp.float32),
            pltpu.VMEM((BPW,), jnp.float32),
            pltpu.VMEM((BPW,), jnp.float32),
            pltpu.VMEM((16,), jnp.float32),
        ] + [pltpu.SemaphoreType.DMA] * (1 + R),
    )
    return kern(u3, i3, Ut, Vt, ubf, vbf, gb1)


def kernel(u, i, U, V, ub, vb, gb):
    u3 = u.reshape(NW, BPW)
    i3 = i.reshape(NW, BPW)
    gb1 = jnp.broadcast_to(jnp.asarray(gb, jnp.float32), (16,))
    return _mf_scores(u3, i3, U.T, V.T,
                      ub.reshape(N_USERS), vb.reshape(N_ITEMS), gb1)
